# baseline (speedup 1.0000x reference)
import jax
import jax.numpy as jnp
from jax import lax
from jax.experimental import pallas as pl
from jax.experimental.pallas import tpu as pltpu
from jax.experimental.pallas import tpu_sc as plsc

B, S = 4096, 200
V, D, C = 100000, 64, 1000
HPAD = 208
NC, NS = 2, 16
NW = NC * NS
RPW = B // NW
L = 16
NBUF = 8
VPT = V // NW
CH = 625

_SC_PARAMS = pltpu.CompilerParams(use_tc_tiling_on_sc=False,
                                  needs_layout_passes=False)


def _cvt_body(w32_hbm, wbf_hbm, fbuf, obuf, sem):
    wid = lax.axis_index("s") * NC + lax.axis_index("c")
    lo = wid * VPT
    iota16 = lax.iota(jnp.int32, L)

    def _fire(c, slot):
        pltpu.async_copy(w32_hbm.at[pl.ds(lo + c * CH, CH)],
                         fbuf.at[slot], sem)

    def _drain(c, slot):
        pltpu.make_async_copy(w32_hbm.at[pl.ds(lo + c * CH, CH)],
                              fbuf.at[slot], sem).wait()

    _fire(0, 0)
    for c in range(VPT // CH):
        if c + 1 < VPT // CH:
            _fire(c + 1, (c + 1) % 2)
        _drain(c, c % 2)
        slot = jnp.int32(c % 2)
        svec = jnp.full((L,), c % 2, jnp.int32)

        def _cv(i, carry):
            r = i >> 1
            off = (i & 1) << 5
            rvec = jnp.full((L,), r, jnp.int32)
            cidx = jnp.full((L,), off, jnp.int32) + 2 * iota16
            a = plsc.load_gather(fbuf, [svec, rvec, cidx])
            bq = plsc.load_gather(fbuf, [svec, rvec, cidx + 1])
            p = plsc.pack(a, bq, format=plsc.PackFormat.INTERLEAVED)
            obuf[r, pl.ds(off, 2 * L)] = p
            return carry
        lax.fori_loop(0, CH * 2, _cv, 0)
        pltpu.sync_copy(obuf, wbf_hbm.at[pl.ds(lo + c * CH, CH)])


@jax.jit
def _sc_cvt(W_emb):
    mesh = plsc.VectorSubcoreMesh(core_axis_name="c", subcore_axis_name="s",
                                  num_cores=NC, num_subcores=NS)
    f = pl.kernel(
        _cvt_body,
        out_type=jax.ShapeDtypeStruct((V, D), jnp.bfloat16),
        mesh=mesh,
        compiler_params=_SC_PARAMS,
        scratch_types=[
            pltpu.VMEM((2, CH, D), jnp.float32),
            pltpu.VMEM((CH, D), jnp.bfloat16),
            pltpu.SemaphoreType.DMA,
        ],
    )
    return f(W_emb)


def _sc_body(x_hbm, emb_hbm, emb_out_hbm, hist_out_hbm,
             xbuf, rows, emb_acc, hist_acc, sem):
    wid = lax.axis_index("s") * NC + lax.axis_index("c")
    base = wid * RPW

    pltpu.sync_copy(x_hbm.at[pl.ds(base * S, RPW * S)], xbuf)

    zeros16 = jnp.zeros((L,), jnp.float32)
    ones16 = jnp.ones((L,), jnp.float32)
    iota16 = lax.iota(jnp.int32, L)

    def _zero_row(r, carry):
        for k in range(HPAD // L):
            hist_acc[r, pl.ds(k * L, L)] = zeros16
        return carry
    lax.fori_loop(0, RPW, _zero_row, 0)

    def _fire(r, buf):
        pltpu.async_copy(emb_hbm.at[xbuf.at[pl.ds(r * S, 128)]],
                         rows.at[buf, pl.ds(0, 128)], sem)
        pltpu.async_copy(emb_hbm.at[xbuf.at[pl.ds(r * S + 128, S - 128)]],
                         rows.at[buf, pl.ds(128, S - 128)], sem)

    def _drain(r, buf):
        pltpu.make_async_copy(emb_hbm.at[xbuf.at[pl.ds(r * S, 128)]],
                              rows.at[buf, pl.ds(0, 128)], sem).wait()
        pltpu.make_async_copy(emb_hbm.at[xbuf.at[pl.ds(r * S + 128, S - 128)]],
                              rows.at[buf, pl.ds(128, S - 128)], sem).wait()

    for rr in range(NBUF):
        _fire(rr, rr)

    def _row(r, carry):
        buf = lax.rem(r, NBUF)

        _drain(r, buf)

        zeros32 = jnp.zeros((2 * L,), jnp.bfloat16)

        def _tok(k, accs):
            a0, a1 = accs
            for u in range(8):
                j = k * 8 + u
                a0 = a0 + rows[buf, j, pl.ds(0, 2 * L)]
                a1 = a1 + rows[buf, j, pl.ds(2 * L, 2 * L)]
            return a0, a1
        a0, a1 = lax.fori_loop(0, S // 8, _tok, (zeros32, zeros32))
        emb_acc[r, pl.ds(0, 2 * L)] = a0
        emb_acc[r, pl.ds(2 * L, 2 * L)] = a1

        rvec = jnp.full((L,), r, jnp.int32)
        xoff = jnp.full((L,), r * S, jnp.int32)
        for g in range((S + L - 1) // L):
            tok = iota16 + (g * L)
            ia = jnp.minimum(tok, S - 1) + xoff
            ib = jnp.minimum(tok + 1, S - 1) + xoff
            a = plsc.load_gather(xbuf, [ia])
            b = plsc.load_gather(xbuf, [ib])
            h = a + 100 * b
            q = (h.astype(jnp.float32) * (1.0 / 199.0)).astype(jnp.int32)
            t = h - q * 199
            t = jnp.where(t < 0, t + 199, t)
            t = jnp.where(t >= 199, t - 199, t) + 1
            t = jnp.where(tok < S - 1, t, 0)
            plsc.addupdate_scatter(hist_acc, [rvec, t], ones16)

        @pl.when(r + NBUF < RPW)
        def _():
            _fire(r + NBUF, buf)
        return carry

    lax.fori_loop(0, RPW, _row, 0)

    pltpu.sync_copy(emb_acc, emb_out_hbm.at[pl.ds(base, RPW)])
    pltpu.sync_copy(hist_acc, hist_out_hbm.at[pl.ds(base, RPW)])


@jax.jit
def _sc_pool(x_flat, W_bf16):
    mesh = plsc.VectorSubcoreMesh(core_axis_name="c", subcore_axis_name="s",
                                  num_cores=NC, num_subcores=NS)
    f = pl.kernel(
        _sc_body,
        out_type=(jax.ShapeDtypeStruct((B, D), jnp.bfloat16),
                  jax.ShapeDtypeStruct((B, HPAD), jnp.float32)),
        mesh=mesh,
        compiler_params=_SC_PARAMS,
        scratch_types=[
            pltpu.VMEM((RPW * S,), jnp.int32),
            pltpu.VMEM((NBUF, S, D), jnp.bfloat16),
            pltpu.VMEM((RPW, D), jnp.bfloat16),
            pltpu.VMEM((RPW, HPAD), jnp.float32),
            pltpu.SemaphoreType.DMA,
        ],
    )
    return f(x_flat, W_bf16)


def _tc_body(emb_ref, hist_ref, wng_ref, fcw_ref, fcb_ref, out_ref):
    emb = emb_ref[...].astype(jnp.float32) * (1.0 / S)
    ng = jax.lax.dot_general(hist_ref[...], wng_ref[...],
                             (((1,), (0,)), ((), ())),
                             preferred_element_type=jnp.float32)
    ng = ng * (1.0 / (S - 1))
    w1 = fcw_ref[:, 0:D]
    w2 = fcw_ref[:, D:2 * D]
    o = jax.lax.dot_general(emb, w1, (((1,), (1,)), ((), ())),
                            preferred_element_type=jnp.float32)
    o = o + jax.lax.dot_general(ng, w2, (((1,), (1,)), ((), ())),
                                preferred_element_type=jnp.float32)
    out_ref[...] = o + fcb_ref[...]


@jax.jit
def _tc_fc(emb_sum, hist, W_ng, fc_w, fc_b):
    BM = 512
    grid = (B // BM,)
    return pl.pallas_call(
        _tc_body,
        grid=grid,
        in_specs=[
            pl.BlockSpec((BM, D), lambda i: (i, 0)),
            pl.BlockSpec((BM, HPAD), lambda i: (i, 0)),
            pl.BlockSpec((HPAD, D), lambda i: (0, 0)),
            pl.BlockSpec((C, 2 * D), lambda i: (0, 0)),
            pl.BlockSpec((1, C), lambda i: (0, 0)),
        ],
        out_specs=pl.BlockSpec((BM, C), lambda i: (i, 0)),
        out_shape=jax.ShapeDtypeStruct((B, C), jnp.float32),
    )(emb_sum, hist, W_ng, fc_w, fc_b)


def kernel(x, W_emb, W_ng, fc_w, fc_b):
    W_bf16 = _sc_cvt(W_emb)
    emb_sum, hist = _sc_pool(x.reshape(B * S), W_bf16)
    return _tc_fc(emb_sum, hist, W_ng[:HPAD], fc_w, fc_b.reshape(1, C))


# speedup vs baseline: 1.0585x; 1.0585x over previous
import jax
import jax.numpy as jnp
from jax import lax
from jax.experimental import pallas as pl
from jax.experimental.pallas import tpu as pltpu
from jax.experimental.pallas import tpu_sc as plsc

B, S = 4096, 200
V, D, C = 100000, 64, 1000
HPAD = 208
NC, NS = 2, 16
NW = NC * NS
RPW = B // NW
L = 16
NBUF = 4

_SC_PARAMS = pltpu.CompilerParams(use_tc_tiling_on_sc=False,
                                  needs_layout_passes=False)


def _sc_body(x_hbm, emb_hbm, emb_out_hbm, hist_out_hbm,
             xbuf, rows, emb_acc, hist_acc, sem):
    wid = lax.axis_index("s") * NC + lax.axis_index("c")
    base = wid * RPW

    pltpu.sync_copy(x_hbm.at[pl.ds(base, RPW)], xbuf)

    zeros16 = jnp.zeros((L,), jnp.float32)
    ones16 = jnp.ones((L,), jnp.float32)
    iota16 = lax.iota(jnp.int32, L)

    def _zero_row(r, carry):
        for k in range(HPAD // L):
            hist_acc[r, pl.ds(k * L, L)] = zeros16
        return carry
    lax.fori_loop(0, RPW, _zero_row, 0)

    def _fire(r, buf):
        pltpu.async_copy(emb_hbm.at[xbuf.at[r, pl.ds(0, 128)]],
                         rows.at[buf, pl.ds(0, 128)], sem)
        pltpu.async_copy(emb_hbm.at[xbuf.at[r, pl.ds(128, S - 128)]],
                         rows.at[buf, pl.ds(128, S - 128)], sem)

    def _drain(r, buf):
        pltpu.make_async_copy(emb_hbm.at[xbuf.at[r, pl.ds(0, 128)]],
                              rows.at[buf, pl.ds(0, 128)], sem).wait()
        pltpu.make_async_copy(emb_hbm.at[xbuf.at[r, pl.ds(128, S - 128)]],
                              rows.at[buf, pl.ds(128, S - 128)], sem).wait()

    for rr in range(NBUF):
        _fire(rr, rr)

    def _row(r, carry):
        buf = lax.rem(r, NBUF)

        _drain(r, buf)

        def _tok(k, accs):
            a0, a1, a2, a3 = accs
            for u in range(8):
                j = k * 8 + u
                a0 = a0 + rows[buf, j, pl.ds(0, L)]
                a1 = a1 + rows[buf, j, pl.ds(L, L)]
                a2 = a2 + rows[buf, j, pl.ds(2 * L, L)]
                a3 = a3 + rows[buf, j, pl.ds(3 * L, L)]
            return a0, a1, a2, a3
        z4 = (zeros16, zeros16, zeros16, zeros16)
        a0, a1, a2, a3 = lax.fori_loop(0, S // 8, _tok, z4)
        emb_acc[r, pl.ds(0, L)] = a0
        emb_acc[r, pl.ds(L, L)] = a1
        emb_acc[r, pl.ds(2 * L, L)] = a2
        emb_acc[r, pl.ds(3 * L, L)] = a3

        rvec = jnp.full((L,), r, jnp.int32)
        for g in range((S + L - 1) // L):
            tok = iota16 + (g * L)
            ia = jnp.minimum(tok, S - 1)
            ib = jnp.minimum(tok + 1, S - 1)
            a = plsc.load_gather(xbuf, [rvec, ia])
            b = plsc.load_gather(xbuf, [rvec, ib])
            h = a + 100 * b
            q = (h.astype(jnp.float32) * (1.0 / 199.0)).astype(jnp.int32)
            t = h - q * 199
            t = jnp.where(t < 0, t + 199, t)
            t = jnp.where(t >= 199, t - 199, t) + 1
            t = jnp.where(tok < S - 1, t, 0)
            plsc.addupdate_scatter(hist_acc, [rvec, t], ones16)

        @pl.when(r + NBUF < RPW)
        def _():
            _fire(r + NBUF, buf)
        return carry

    lax.fori_loop(0, RPW, _row, 0)

    pltpu.sync_copy(emb_acc, emb_out_hbm.at[pl.ds(base, RPW)])
    pltpu.sync_copy(hist_acc, hist_out_hbm.at[pl.ds(base, RPW)])


@jax.jit
def _sc_pool(x, W_emb):
    mesh = plsc.VectorSubcoreMesh(core_axis_name="c", subcore_axis_name="s",
                                  num_cores=NC, num_subcores=NS)
    f = pl.kernel(
        _sc_body,
        out_type=(jax.ShapeDtypeStruct((B, D), jnp.float32),
                  jax.ShapeDtypeStruct((B, HPAD), jnp.float32)),
        mesh=mesh,
        compiler_params=_SC_PARAMS,
        scratch_types=[
            pltpu.VMEM((RPW, S), jnp.int32),
            pltpu.VMEM((NBUF, S, D), jnp.float32),
            pltpu.VMEM((RPW, D), jnp.float32),
            pltpu.VMEM((RPW, HPAD), jnp.float32),
            pltpu.SemaphoreType.DMA,
        ],
    )
    return f(x, W_emb)


def _tc_body(emb_ref, hist_ref, wng_ref, fcw_ref, fcb_ref, out_ref):
    emb = emb_ref[...] * (1.0 / S)
    ng = jax.lax.dot_general(hist_ref[...], wng_ref[...],
                             (((1,), (0,)), ((), ())),
                             preferred_element_type=jnp.float32)
    ng = ng * (1.0 / (S - 1))
    w1 = fcw_ref[:, 0:D]
    w2 = fcw_ref[:, D:2 * D]
    o = jax.lax.dot_general(emb, w1, (((1,), (1,)), ((), ())),
                            preferred_element_type=jnp.float32)
    o = o + jax.lax.dot_general(ng, w2, (((1,), (1,)), ((), ())),
                                preferred_element_type=jnp.float32)
    out_ref[...] = o + fcb_ref[...]


@jax.jit
def _tc_fc(emb_sum, hist, W_ng, fc_w, fc_b):
    BM = 512
    grid = (B // BM,)
    return pl.pallas_call(
        _tc_body,
        grid=grid,
        in_specs=[
            pl.BlockSpec((BM, D), lambda i: (i, 0)),
            pl.BlockSpec((BM, HPAD), lambda i: (i, 0)),
            pl.BlockSpec((HPAD, D), lambda i: (0, 0)),
            pl.BlockSpec((C, 2 * D), lambda i: (0, 0)),
            pl.BlockSpec((1, C), lambda i: (0, 0)),
        ],
        out_specs=pl.BlockSpec((BM, C), lambda i: (i, 0)),
        out_shape=jax.ShapeDtypeStruct((B, C), jnp.float32),
    )(emb_sum, hist, W_ng, fc_w, fc_b)


def kernel(x, W_emb, W_ng, fc_w, fc_b):
    emb_sum, hist = _sc_pool(x, W_emb)
    return _tc_fc(emb_sum, hist, W_ng[:HPAD], fc_w, fc_b.reshape(1, C))


# speedup vs baseline: 1.0602x; 1.0016x over previous
import jax
import jax.numpy as jnp
from jax import lax
from jax.experimental import pallas as pl
from jax.experimental.pallas import tpu as pltpu
from jax.experimental.pallas import tpu_sc as plsc

B, S = 4096, 200
V, D, C = 100000, 64, 1000
HPAD = 208
NC, NS = 2, 16
NW = NC * NS
RPW = B // NW
L = 16
NBUF = 4

_SC_PARAMS = pltpu.CompilerParams(use_tc_tiling_on_sc=False,
                                  needs_layout_passes=False)


def _sc_body(x_hbm, emb_hbm, emb_out_hbm, hist_out_hbm,
             xbuf, rows, emb_acc, hist_acc, sem):
    wid = lax.axis_index("s") * NC + lax.axis_index("c")
    base = wid * RPW

    pltpu.sync_copy(x_hbm.at[pl.ds(base, RPW)], xbuf)

    zeros16 = jnp.zeros((L,), jnp.float32)
    ones16 = jnp.ones((L,), jnp.float32)
    iota16 = lax.iota(jnp.int32, L)

    def _zero_row(r, carry):
        for k in range(HPAD // L):
            hist_acc[r, pl.ds(k * L, L)] = zeros16
        return carry
    lax.fori_loop(0, RPW, _zero_row, 0)

    def _fire(r, buf):
        pltpu.async_copy(emb_hbm.at[xbuf.at[r, pl.ds(0, 128)]],
                         rows.at[buf, pl.ds(0, 128)], sem)
        pltpu.async_copy(emb_hbm.at[xbuf.at[r, pl.ds(128, S - 128)]],
                         rows.at[buf, pl.ds(128, S - 128)], sem)

    def _drain(r, buf):
        pltpu.make_async_copy(emb_hbm.at[xbuf.at[r, pl.ds(0, 128)]],
                              rows.at[buf, pl.ds(0, 128)], sem).wait()
        pltpu.make_async_copy(emb_hbm.at[xbuf.at[r, pl.ds(128, S - 128)]],
                              rows.at[buf, pl.ds(128, S - 128)], sem).wait()

    for rr in range(NBUF):
        _fire(rr, rr)

    def _row(r, carry):
        buf = lax.rem(r, NBUF)

        _drain(r, buf)

        def _tok(k, accs):
            a0, a1, a2, a3 = accs
            for u in range(8):
                j = k * 8 + u
                a0 = a0 + rows[buf, j, pl.ds(0, L)]
                a1 = a1 + rows[buf, j, pl.ds(L, L)]
                a2 = a2 + rows[buf, j, pl.ds(2 * L, L)]
                a3 = a3 + rows[buf, j, pl.ds(3 * L, L)]
            return a0, a1, a2, a3
        z4 = (zeros16, zeros16, zeros16, zeros16)
        a0, a1, a2, a3 = lax.fori_loop(0, S // 8, _tok, z4)
        emb_acc[r, pl.ds(0, L)] = a0
        emb_acc[r, pl.ds(L, L)] = a1
        emb_acc[r, pl.ds(2 * L, L)] = a2
        emb_acc[r, pl.ds(3 * L, L)] = a3

        rvec = jnp.full((L,), r, jnp.int32)
        for g in range((S + L - 1) // L):
            tok = iota16 + (g * L)
            ia = jnp.minimum(tok, S - 1)
            ib = jnp.minimum(tok + 1, S - 1)
            a = plsc.load_gather(xbuf, [rvec, ia])
            b = plsc.load_gather(xbuf, [rvec, ib])
            h = a + 100 * b
            q = (h.astype(jnp.float32) * (1.0 / 199.0)).astype(jnp.int32)
            t = h - q * 199
            t = jnp.where(t < 0, t + 199, t)
            t = jnp.where(t >= 199, t - 199, t) + 1
            t = jnp.where(tok < S - 1, t, 0)
            plsc.addupdate_scatter(hist_acc, [rvec, t], ones16)

        @pl.when(r + NBUF < RPW)
        def _():
            _fire(r + NBUF, buf)
        return carry

    lax.fori_loop(0, RPW, _row, 0)

    pltpu.sync_copy(emb_acc, emb_out_hbm.at[pl.ds(base, RPW)])
    pltpu.sync_copy(hist_acc, hist_out_hbm.at[pl.ds(base, RPW)])


def _sc_pool(x, W_emb):
    mesh = plsc.VectorSubcoreMesh(core_axis_name="c", subcore_axis_name="s",
                                  num_cores=NC, num_subcores=NS)
    f = pl.kernel(
        _sc_body,
        out_type=(jax.ShapeDtypeStruct((B, D), jnp.float32),
                  jax.ShapeDtypeStruct((B, HPAD), jnp.float32)),
        mesh=mesh,
        compiler_params=_SC_PARAMS,
        scratch_types=[
            pltpu.VMEM((RPW, S), jnp.int32),
            pltpu.VMEM((NBUF, S, D), jnp.float32),
            pltpu.VMEM((RPW, D), jnp.float32),
            pltpu.VMEM((RPW, HPAD), jnp.float32),
            pltpu.SemaphoreType.DMA,
        ],
    )
    return f(x, W_emb)


def _tc_body(emb_ref, hist_ref, wng_ref, fcw_ref, fcb_ref, out_ref):
    emb = emb_ref[...] * (1.0 / S)
    ng = jax.lax.dot_general(hist_ref[...], wng_ref[...],
                             (((1,), (0,)), ((), ())),
                             preferred_element_type=jnp.float32)
    ng = ng * (1.0 / (S - 1))
    w1 = fcw_ref[:, 0:D]
    w2 = fcw_ref[:, D:2 * D]
    o = jax.lax.dot_general(emb, w1, (((1,), (1,)), ((), ())),
                            preferred_element_type=jnp.float32)
    o = o + jax.lax.dot_general(ng, w2, (((1,), (1,)), ((), ())),
                                preferred_element_type=jnp.float32)
    out_ref[...] = o + fcb_ref[...]


def _tc_fc(emb_sum, hist, W_ng, fc_w, fc_b):
    BM = 512
    grid = (B // BM,)
    return pl.pallas_call(
        _tc_body,
        grid=grid,
        in_specs=[
            pl.BlockSpec((BM, D), lambda i: (i, 0)),
            pl.BlockSpec((BM, HPAD), lambda i: (i, 0)),
            pl.BlockSpec((HPAD, D), lambda i: (0, 0)),
            pl.BlockSpec((C, 2 * D), lambda i: (0, 0)),
            pl.BlockSpec((1, C), lambda i: (0, 0)),
        ],
        out_specs=pl.BlockSpec((BM, C), lambda i: (i, 0)),
        out_shape=jax.ShapeDtypeStruct((B, C), jnp.float32),
    )(emb_sum, hist, W_ng, fc_w, fc_b)


def kernel(x, W_emb, W_ng, fc_w, fc_b):
    emb_sum, hist = _sc_pool(x, W_emb)
    return _tc_fc(emb_sum, hist, W_ng[:HPAD], fc_w, fc_b.reshape(1, C))
